# baseline (speedup 1.0000x reference)
import functools

import jax
import jax.numpy as jnp
from jax import lax
from jax.experimental import pallas as pl
from jax.experimental.pallas import tpu as pltpu
from jax.experimental.pallas import tpu_sc as plsc

N_NODES = 100000
N_EDGES = 6400000
NUM_GRAPHS = 1024

NC = 2
NS = 16
L = 16
W = 16
NW = NC * NS
EPW = N_EDGES // NW
CHUNK = 2000
NCHUNK = EPW // CHUNK
CVECS = CHUNK // L


def _edge_var_sc(tbl, src, dst):
    mesh = plsc.VectorSubcoreMesh(
        core_axis_name="c", subcore_axis_name="s", num_cores=NC, num_subcores=NS
    )

    @functools.partial(
        pl.kernel,
        out_type=[
            jax.ShapeDtypeStruct((NW, NUM_GRAPHS), jnp.float32),
            jax.ShapeDtypeStruct((NW, NUM_GRAPHS), jnp.float32),
        ],
        mesh=mesh,
        scratch_types=[
            pltpu.VMEM((CHUNK,), jnp.int32),
            pltpu.VMEM((CHUNK,), jnp.int32),
            pltpu.VMEM((CHUNK, W), jnp.float32),
            pltpu.VMEM((CHUNK, W), jnp.float32),
            pltpu.VMEM((NUM_GRAPHS,), jnp.float32),
            pltpu.VMEM((NUM_GRAPHS,), jnp.float32),
            pltpu.SemaphoreType.DMA,
        ],
        compiler_params=pltpu.CompilerParams(
            needs_layout_passes=False, use_tc_tiling_on_sc=False
        ),
    )
    def body(tbl_hbm, src_hbm, dst_hbm, sums_out, cnts_out,
             sidx, didx, srows, drows, lsum, lcnt, sem):
        cid = lax.axis_index("c")
        sid = lax.axis_index("s")
        wid = cid * NS + sid

        def zbody(i, _):
            off = pl.multiple_of(i * L, L)
            lsum[pl.ds(off, L)] = jnp.zeros((L,), jnp.float32)
            lcnt[pl.ds(off, L)] = jnp.zeros((L,), jnp.float32)
            return 0

        lax.fori_loop(0, NUM_GRAPHS // L, zbody, 0)

        ones = jnp.ones((L,), jnp.float32)
        c0 = jnp.zeros((L,), jnp.int32)
        c1 = jnp.full((L,), 1, jnp.int32)
        c2 = jnp.full((L,), 2, jnp.int32)
        c3 = jnp.full((L,), 3, jnp.int32)

        def vec_body(vi, _):
            rows = vi * L + lax.iota(jnp.int32, L)
            sx = plsc.load_gather(srows, [rows, c0])
            sy = plsc.load_gather(srows, [rows, c1])
            sz = plsc.load_gather(srows, [rows, c2])
            bf = plsc.load_gather(srows, [rows, c3])
            dx = plsc.load_gather(drows, [rows, c0])
            dy = plsc.load_gather(drows, [rows, c1])
            dz = plsc.load_gather(drows, [rows, c2])
            ex = dx - sx
            ey = dy - sy
            ez = dz - sz
            s = ex * ex + ey * ey + ez * ez + jnp.float32(1e-12)
            bits = plsc.bitcast(s, jnp.int32)
            bits = jnp.int32(0x5F3759DF) - lax.shift_right_logical(bits, 1)
            y = plsc.bitcast(bits, jnp.float32)
            half = s * jnp.float32(0.5)
            for _ in range(3):
                y = y * (jnp.float32(1.5) - half * y * y)
            eu = s * y
            d = eu - jnp.float32(1.0)
            var = d * d
            bidx = bf.astype(jnp.int32)
            plsc.addupdate_scatter(lsum, [bidx], var)
            plsc.addupdate_scatter(lcnt, [bidx], ones)
            return 0

        def chunk_body(ci, _):
            base = pl.multiple_of(wid * EPW + ci * CHUNK, 8)
            pltpu.sync_copy(src_hbm.at[pl.ds(base, CHUNK)], sidx)
            pltpu.sync_copy(dst_hbm.at[pl.ds(base, CHUNK)], didx)
            cp1 = pltpu.async_copy(tbl_hbm.at[sidx], srows, sem)
            cp2 = pltpu.async_copy(tbl_hbm.at[didx], drows, sem)
            cp1.wait()
            cp2.wait()
            lax.fori_loop(0, CVECS, vec_body, 0)
            return 0

        lax.fori_loop(0, NCHUNK, chunk_body, 0)

        pltpu.sync_copy(lsum, sums_out.at[wid])
        pltpu.sync_copy(lcnt, cnts_out.at[wid])

    return body(tbl, src, dst)


def _finalize_tc(sums_p, cnts_p):
    def tc_body(s_ref, c_ref, o_ref):
        s = jnp.sum(s_ref[...], axis=0)
        c = jnp.sum(c_ref[...], axis=0)
        gv = s / jnp.maximum(c, 1.0)
        o_ref[...] = (jnp.sum(gv) / jnp.float32(NUM_GRAPHS)).reshape(1, 1)

    out = pl.pallas_call(
        tc_body,
        out_shape=jax.ShapeDtypeStruct((1, 1), jnp.float32),
    )(sums_p, cnts_p)
    return out[0, 0]


def kernel(node_pos, edge_index, batch_ids):
    bi = batch_ids.astype(jnp.int32)
    tbl = jnp.concatenate(
        [
            node_pos,
            bi.astype(jnp.float32)[:, None],
            jnp.zeros((N_NODES, W - 4), jnp.float32),
        ],
        axis=1,
    )
    ei = edge_index.astype(jnp.int32)
    src = ei[0]
    dst = ei[1]
    sums_p, cnts_p = _edge_var_sc(tbl, src, dst)
    return _finalize_tc(sums_p, cnts_p)


# speedup vs baseline: 1.6518x; 1.6518x over previous
import functools

import jax
import jax.numpy as jnp
from jax import lax
from jax.experimental import pallas as pl
from jax.experimental.pallas import tpu as pltpu
from jax.experimental.pallas import tpu_sc as plsc

N_NODES = 100000
N_EDGES = 6400000
NUM_GRAPHS = 1024

NC = 2
NS = 16
L = 16
W = 8
NW = NC * NS
EPW = N_EDGES // NW
CHUNK = 2000
NCHUNK = EPW // CHUNK
CVECS = CHUNK // L


def _edge_var_sc(tbl, src, dst):
    mesh = plsc.VectorSubcoreMesh(
        core_axis_name="c", subcore_axis_name="s", num_cores=NC, num_subcores=NS
    )

    @functools.partial(
        pl.kernel,
        out_type=[
            jax.ShapeDtypeStruct((NW, NUM_GRAPHS), jnp.float32),
            jax.ShapeDtypeStruct((NW, NUM_GRAPHS), jnp.float32),
        ],
        mesh=mesh,
        scratch_types=[
            pltpu.VMEM_SHARED((N_NODES, W), jnp.float32),
            pltpu.VMEM((CHUNK,), jnp.int32),
            pltpu.VMEM((CHUNK,), jnp.int32),
            pltpu.VMEM((CHUNK, W), jnp.float32),
            pltpu.VMEM((CHUNK, W), jnp.float32),
            pltpu.VMEM((NUM_GRAPHS,), jnp.float32),
            pltpu.VMEM((NUM_GRAPHS,), jnp.float32),
            pltpu.SemaphoreType.DMA,
        ],
        compiler_params=pltpu.CompilerParams(
            needs_layout_passes=False, use_tc_tiling_on_sc=False
        ),
    )
    def body(tbl_hbm, src_hbm, dst_hbm, sums_out, cnts_out,
             tbl_sh, sidx, didx, srows, drows, lsum, lcnt, sem):
        cid = lax.axis_index("c")
        sid = lax.axis_index("s")
        wid = cid * NS + sid

        @pl.when(sid == 0)
        def _():
            pltpu.sync_copy(tbl_hbm, tbl_sh)

        def zbody(i, _):
            off = pl.multiple_of(i * L, L)
            lsum[pl.ds(off, L)] = jnp.zeros((L,), jnp.float32)
            lcnt[pl.ds(off, L)] = jnp.zeros((L,), jnp.float32)
            return 0

        lax.fori_loop(0, NUM_GRAPHS // L, zbody, 0)
        plsc.subcore_barrier()

        ones = jnp.ones((L,), jnp.float32)
        c0 = jnp.zeros((L,), jnp.int32)
        c1 = jnp.full((L,), 1, jnp.int32)
        c2 = jnp.full((L,), 2, jnp.int32)
        c3 = jnp.full((L,), 3, jnp.int32)

        def vec_body(vi, _):
            rows = vi * L + lax.iota(jnp.int32, L)
            sx = plsc.load_gather(srows, [rows, c0])
            sy = plsc.load_gather(srows, [rows, c1])
            sz = plsc.load_gather(srows, [rows, c2])
            bf = plsc.load_gather(srows, [rows, c3])
            dx = plsc.load_gather(drows, [rows, c0])
            dy = plsc.load_gather(drows, [rows, c1])
            dz = plsc.load_gather(drows, [rows, c2])
            ex = dx - sx
            ey = dy - sy
            ez = dz - sz
            s = ex * ex + ey * ey + ez * ez + jnp.float32(1e-12)
            bits = plsc.bitcast(s, jnp.int32)
            bits = jnp.int32(0x5F3759DF) - lax.shift_right_logical(bits, 1)
            y = plsc.bitcast(bits, jnp.float32)
            half = s * jnp.float32(0.5)
            for _ in range(3):
                y = y * (jnp.float32(1.5) - half * y * y)
            eu = s * y
            d = eu - jnp.float32(1.0)
            var = d * d
            bidx = bf.astype(jnp.int32)
            plsc.addupdate_scatter(lsum, [bidx], var)
            plsc.addupdate_scatter(lcnt, [bidx], ones)
            return 0

        def chunk_body(ci, _):
            base = pl.multiple_of(wid * EPW + ci * CHUNK, 8)
            pltpu.sync_copy(src_hbm.at[pl.ds(base, CHUNK)], sidx)
            pltpu.sync_copy(dst_hbm.at[pl.ds(base, CHUNK)], didx)
            cp1 = pltpu.async_copy(tbl_sh.at[sidx], srows, sem)
            cp2 = pltpu.async_copy(tbl_sh.at[didx], drows, sem)
            cp1.wait()
            cp2.wait()
            lax.fori_loop(0, CVECS, vec_body, 0)
            return 0

        lax.fori_loop(0, NCHUNK, chunk_body, 0)

        pltpu.sync_copy(lsum, sums_out.at[wid])
        pltpu.sync_copy(lcnt, cnts_out.at[wid])

    return body(tbl, src, dst)


def _finalize_tc(sums_p, cnts_p):
    def tc_body(s_ref, c_ref, o_ref):
        s = jnp.sum(s_ref[...], axis=0)
        c = jnp.sum(c_ref[...], axis=0)
        gv = s / jnp.maximum(c, 1.0)
        o_ref[...] = (jnp.sum(gv) / jnp.float32(NUM_GRAPHS)).reshape(1, 1)

    out = pl.pallas_call(
        tc_body,
        out_shape=jax.ShapeDtypeStruct((1, 1), jnp.float32),
    )(sums_p, cnts_p)
    return out[0, 0]


def kernel(node_pos, edge_index, batch_ids):
    bi = batch_ids.astype(jnp.int32)
    tbl = jnp.concatenate(
        [
            node_pos,
            bi.astype(jnp.float32)[:, None],
            jnp.zeros((N_NODES, W - 4), jnp.float32),
        ],
        axis=1,
    )
    ei = edge_index.astype(jnp.int32)
    src = ei[0]
    dst = ei[1]
    sums_p, cnts_p = _edge_var_sc(tbl, src, dst)
    return _finalize_tc(sums_p, cnts_p)


# speedup vs baseline: 2.3130x; 1.4003x over previous
import functools

import jax
import jax.numpy as jnp
from jax import lax
from jax.experimental import pallas as pl
from jax.experimental.pallas import tpu as pltpu
from jax.experimental.pallas import tpu_sc as plsc

N_NODES = 100000
N_EDGES = 6400000
NUM_GRAPHS = 1024

NC = 2
NS = 16
L = 16
W = 8
NW = NC * NS
EPW = N_EDGES // NW
CHUNK = 2000
NCHUNK = EPW // CHUNK
CVECS = CHUNK // L


def _edge_var_sc(tbl, src, dst):
    mesh = plsc.VectorSubcoreMesh(
        core_axis_name="c", subcore_axis_name="s", num_cores=NC, num_subcores=NS
    )

    @functools.partial(
        pl.kernel,
        out_type=[
            jax.ShapeDtypeStruct((NW, NUM_GRAPHS), jnp.float32),
            jax.ShapeDtypeStruct((NW, NUM_GRAPHS), jnp.float32),
        ],
        mesh=mesh,
        scratch_types=[
            pltpu.VMEM_SHARED((N_NODES, W), jnp.float32),
            pltpu.VMEM((CHUNK,), jnp.int32),
            pltpu.VMEM((CHUNK,), jnp.int32),
            pltpu.VMEM((CHUNK,), jnp.int32),
            pltpu.VMEM((CHUNK,), jnp.int32),
            pltpu.VMEM((CHUNK, W), jnp.float32),
            pltpu.VMEM((CHUNK, W), jnp.float32),
            pltpu.VMEM((CHUNK, W), jnp.float32),
            pltpu.VMEM((CHUNK, W), jnp.float32),
            pltpu.VMEM((NUM_GRAPHS,), jnp.float32),
            pltpu.VMEM((NUM_GRAPHS,), jnp.float32),
            pltpu.SemaphoreType.DMA,
            pltpu.SemaphoreType.DMA,
            pltpu.SemaphoreType.DMA,
            pltpu.SemaphoreType.DMA,
        ],
        compiler_params=pltpu.CompilerParams(
            needs_layout_passes=False, use_tc_tiling_on_sc=False
        ),
    )
    def body(tbl_hbm, src_hbm, dst_hbm, sums_out, cnts_out,
             tbl_sh, sidx0, didx0, sidx1, didx1, srows0, drows0,
             srows1, drows1, lsum, lcnt, semi0, semi1, semg0, semg1):
        cid = lax.axis_index("c")
        sid = lax.axis_index("s")
        wid = cid * NS + sid

        @pl.when(sid == 0)
        def _():
            pltpu.sync_copy(tbl_hbm, tbl_sh)

        def zbody(i, _):
            off = pl.multiple_of(i * L, L)
            lsum[pl.ds(off, L)] = jnp.zeros((L,), jnp.float32)
            lcnt[pl.ds(off, L)] = jnp.zeros((L,), jnp.float32)
            return 0

        lax.fori_loop(0, NUM_GRAPHS // L, zbody, 0)
        plsc.subcore_barrier()

        ones = jnp.ones((L,), jnp.float32)
        c0 = jnp.zeros((L,), jnp.int32)
        c1 = jnp.full((L,), 1, jnp.int32)
        c2 = jnp.full((L,), 2, jnp.int32)
        c3 = jnp.full((L,), 3, jnp.int32)

        def compute(srows, drows):
            def vec_body(vi, _):
                rows = vi * L + lax.iota(jnp.int32, L)
                sx = plsc.load_gather(srows, [rows, c0])
                sy = plsc.load_gather(srows, [rows, c1])
                sz = plsc.load_gather(srows, [rows, c2])
                bf = plsc.load_gather(srows, [rows, c3])
                dx = plsc.load_gather(drows, [rows, c0])
                dy = plsc.load_gather(drows, [rows, c1])
                dz = plsc.load_gather(drows, [rows, c2])
                ex = dx - sx
                ey = dy - sy
                ez = dz - sz
                s = ex * ex + ey * ey + ez * ez + jnp.float32(1e-12)
                bits = plsc.bitcast(s, jnp.int32)
                bits = jnp.int32(0x5F3759DF) - lax.shift_right_logical(bits, 1)
                y = plsc.bitcast(bits, jnp.float32)
                half = s * jnp.float32(0.5)
                for _ in range(3):
                    y = y * (jnp.float32(1.5) - half * y * y)
                eu = s * y
                d = eu - jnp.float32(1.0)
                var = d * d
                bidx = bf.astype(jnp.int32)
                plsc.addupdate_scatter(lsum, [bidx], var)
                plsc.addupdate_scatter(lcnt, [bidx], ones)
                return 0

            lax.fori_loop(0, CVECS, vec_body, 0)

        def launch_idx(ci, sidx, didx, semi):
            base = pl.multiple_of(wid * EPW + ci * CHUNK, 8)
            pltpu.async_copy(src_hbm.at[pl.ds(base, CHUNK)], sidx, semi)
            pltpu.async_copy(dst_hbm.at[pl.ds(base, CHUNK)], didx, semi)

        def wait_idx(sidx, didx, semi):
            pltpu.make_async_copy(src_hbm.at[pl.ds(0, CHUNK)], sidx, semi).wait()
            pltpu.make_async_copy(dst_hbm.at[pl.ds(0, CHUNK)], didx, semi).wait()

        def launch_gather(sidx, didx, srows, drows, semg):
            pltpu.async_copy(tbl_sh.at[sidx], srows, semg)
            pltpu.async_copy(tbl_sh.at[didx], drows, semg)

        def wait_gather(sidx, didx, srows, drows, semg):
            pltpu.make_async_copy(tbl_hbm.at[sidx], srows, semg).wait()
            pltpu.make_async_copy(tbl_hbm.at[didx], drows, semg).wait()

        launch_idx(0, sidx0, didx0, semi0)
        wait_idx(sidx0, didx0, semi0)
        launch_gather(sidx0, didx0, srows0, drows0, semg0)
        launch_idx(1, sidx1, didx1, semi1)

        def pair_body(g, _):
            ca = 2 * g
            wait_gather(sidx0, didx0, srows0, drows0, semg0)
            wait_idx(sidx1, didx1, semi1)
            launch_gather(sidx1, didx1, srows1, drows1, semg1)

            @pl.when(ca + 2 < NCHUNK)
            def _():
                launch_idx(ca + 2, sidx0, didx0, semi0)

            compute(srows0, drows0)

            cb = ca + 1
            wait_gather(sidx1, didx1, srows1, drows1, semg1)

            @pl.when(cb + 1 < NCHUNK)
            def _():
                wait_idx(sidx0, didx0, semi0)
                launch_gather(sidx0, didx0, srows0, drows0, semg0)

            @pl.when(cb + 2 < NCHUNK)
            def _():
                launch_idx(cb + 2, sidx1, didx1, semi1)

            compute(srows1, drows1)
            return 0

        lax.fori_loop(0, NCHUNK // 2, pair_body, 0)

        pltpu.sync_copy(lsum, sums_out.at[wid])
        pltpu.sync_copy(lcnt, cnts_out.at[wid])

    return body(tbl, src, dst)


def _finalize_tc(sums_p, cnts_p):
    def tc_body(s_ref, c_ref, o_ref):
        s = jnp.sum(s_ref[...], axis=0)
        c = jnp.sum(c_ref[...], axis=0)
        gv = s / jnp.maximum(c, 1.0)
        o_ref[...] = (jnp.sum(gv) / jnp.float32(NUM_GRAPHS)).reshape(1, 1)

    out = pl.pallas_call(
        tc_body,
        out_shape=jax.ShapeDtypeStruct((1, 1), jnp.float32),
    )(sums_p, cnts_p)
    return out[0, 0]


def kernel(node_pos, edge_index, batch_ids):
    bi = batch_ids.astype(jnp.int32)
    tbl = jnp.concatenate(
        [
            node_pos,
            bi.astype(jnp.float32)[:, None],
            jnp.zeros((N_NODES, W - 4), jnp.float32),
        ],
        axis=1,
    )
    ei = edge_index.astype(jnp.int32)
    src = ei[0]
    dst = ei[1]
    sums_p, cnts_p = _edge_var_sc(tbl, src, dst)
    return _finalize_tc(sums_p, cnts_p)


# speedup vs baseline: 2.4251x; 1.0485x over previous
import functools

import jax
import jax.numpy as jnp
from jax import lax
from jax.experimental import pallas as pl
from jax.experimental.pallas import tpu as pltpu
from jax.experimental.pallas import tpu_sc as plsc

N_NODES = 100000
N_EDGES = 6400000
NUM_GRAPHS = 1024

NC = 2
NS = 16
L = 16
W = 8
NW = NC * NS
EPW = N_EDGES // NW
CHUNK = 2000
NCHUNK = EPW // CHUNK
CVECS = CHUNK // L


def _edge_var_sc(tbl, ei):
    mesh = plsc.VectorSubcoreMesh(
        core_axis_name="c", subcore_axis_name="s", num_cores=NC, num_subcores=NS
    )

    @functools.partial(
        pl.kernel,
        out_type=[
            jax.ShapeDtypeStruct((NW, NUM_GRAPHS), jnp.float32),
            jax.ShapeDtypeStruct((NW, NUM_GRAPHS), jnp.float32),
        ],
        mesh=mesh,
        scratch_types=[
            pltpu.VMEM_SHARED((N_NODES, W), jnp.float32),
            pltpu.VMEM((CHUNK,), jnp.int32),
            pltpu.VMEM((CHUNK,), jnp.int32),
            pltpu.VMEM((CHUNK,), jnp.int32),
            pltpu.VMEM((CHUNK,), jnp.int32),
            pltpu.VMEM((CHUNK, W), jnp.float32),
            pltpu.VMEM((CHUNK, W), jnp.float32),
            pltpu.VMEM((CHUNK, W), jnp.float32),
            pltpu.VMEM((CHUNK, W), jnp.float32),
            pltpu.VMEM((NUM_GRAPHS,), jnp.float32),
            pltpu.VMEM((NUM_GRAPHS,), jnp.float32),
            pltpu.SemaphoreType.DMA,
            pltpu.SemaphoreType.DMA,
            pltpu.SemaphoreType.DMA,
            pltpu.SemaphoreType.DMA,
        ],
        compiler_params=pltpu.CompilerParams(
            needs_layout_passes=False, use_tc_tiling_on_sc=False
        ),
    )
    def body(tbl_hbm, ei_hbm, sums_out, cnts_out,
             tbl_sh, sidx0, didx0, sidx1, didx1, srows0, drows0,
             srows1, drows1, lsum, lcnt, semi0, semi1, semg0, semg1):
        cid = lax.axis_index("c")
        sid = lax.axis_index("s")
        wid = cid * NS + sid

        @pl.when(sid == 0)
        def _():
            pltpu.sync_copy(tbl_hbm, tbl_sh)

        def zbody(i, _):
            off = pl.multiple_of(i * L, L)
            lsum[pl.ds(off, L)] = jnp.zeros((L,), jnp.float32)
            lcnt[pl.ds(off, L)] = jnp.zeros((L,), jnp.float32)
            return 0

        lax.fori_loop(0, NUM_GRAPHS // L, zbody, 0)
        plsc.subcore_barrier()

        ones = jnp.ones((L,), jnp.float32)
        c0 = jnp.zeros((L,), jnp.int32)
        c1 = jnp.full((L,), 1, jnp.int32)
        c2 = jnp.full((L,), 2, jnp.int32)
        c3 = jnp.full((L,), 3, jnp.int32)

        def compute(srows, drows):
            def vec_body(vi, _):
                rows = vi * L + lax.iota(jnp.int32, L)
                sx = plsc.load_gather(srows, [rows, c0])
                sy = plsc.load_gather(srows, [rows, c1])
                sz = plsc.load_gather(srows, [rows, c2])
                bf = plsc.load_gather(srows, [rows, c3])
                dx = plsc.load_gather(drows, [rows, c0])
                dy = plsc.load_gather(drows, [rows, c1])
                dz = plsc.load_gather(drows, [rows, c2])
                ex = dx - sx
                ey = dy - sy
                ez = dz - sz
                s = ex * ex + ey * ey + ez * ez + jnp.float32(1e-12)
                bits = plsc.bitcast(s, jnp.int32)
                bits = jnp.int32(0x5F3759DF) - lax.shift_right_logical(bits, 1)
                y = plsc.bitcast(bits, jnp.float32)
                half = s * jnp.float32(0.5)
                for _ in range(3):
                    y = y * (jnp.float32(1.5) - half * y * y)
                eu = s * y
                d = eu - jnp.float32(1.0)
                var = d * d
                bidx = bf.astype(jnp.int32)
                plsc.addupdate_scatter(lsum, [bidx], var)
                plsc.addupdate_scatter(lcnt, [bidx], ones)
                return 0

            lax.fori_loop(0, CVECS, vec_body, 0)

        def launch_idx(ci, sidx, didx, semi):
            base = pl.multiple_of(wid * EPW + ci * CHUNK, 8)
            pltpu.async_copy(ei_hbm.at[0, pl.ds(base, CHUNK)], sidx, semi)
            pltpu.async_copy(ei_hbm.at[1, pl.ds(base, CHUNK)], didx, semi)

        def wait_idx(sidx, didx, semi):
            pltpu.make_async_copy(ei_hbm.at[0, pl.ds(0, CHUNK)], sidx, semi).wait()
            pltpu.make_async_copy(ei_hbm.at[1, pl.ds(0, CHUNK)], didx, semi).wait()

        def launch_gather(sidx, didx, srows, drows, semg):
            pltpu.async_copy(tbl_sh.at[sidx], srows, semg)
            pltpu.async_copy(tbl_sh.at[didx], drows, semg)

        def wait_gather(sidx, didx, srows, drows, semg):
            pltpu.make_async_copy(tbl_hbm.at[sidx], srows, semg).wait()
            pltpu.make_async_copy(tbl_hbm.at[didx], drows, semg).wait()

        launch_idx(0, sidx0, didx0, semi0)
        wait_idx(sidx0, didx0, semi0)
        launch_gather(sidx0, didx0, srows0, drows0, semg0)
        launch_idx(1, sidx1, didx1, semi1)

        def pair_body(g, _):
            ca = 2 * g
            wait_gather(sidx0, didx0, srows0, drows0, semg0)
            wait_idx(sidx1, didx1, semi1)
            launch_gather(sidx1, didx1, srows1, drows1, semg1)

            @pl.when(ca + 2 < NCHUNK)
            def _():
                launch_idx(ca + 2, sidx0, didx0, semi0)

            compute(srows0, drows0)

            cb = ca + 1
            wait_gather(sidx1, didx1, srows1, drows1, semg1)

            @pl.when(cb + 1 < NCHUNK)
            def _():
                wait_idx(sidx0, didx0, semi0)
                launch_gather(sidx0, didx0, srows0, drows0, semg0)

            @pl.when(cb + 2 < NCHUNK)
            def _():
                launch_idx(cb + 2, sidx1, didx1, semi1)

            compute(srows1, drows1)
            return 0

        lax.fori_loop(0, NCHUNK // 2, pair_body, 0)

        pltpu.sync_copy(lsum, sums_out.at[wid])
        pltpu.sync_copy(lcnt, cnts_out.at[wid])

    return body(tbl, ei)


def _finalize_tc(sums_p, cnts_p):
    def tc_body(s_ref, c_ref, o_ref):
        s = jnp.sum(s_ref[...], axis=0)
        c = jnp.sum(c_ref[...], axis=0)
        gv = s / jnp.maximum(c, 1.0)
        o_ref[...] = (jnp.sum(gv) / jnp.float32(NUM_GRAPHS)).reshape(1, 1)

    out = pl.pallas_call(
        tc_body,
        out_shape=jax.ShapeDtypeStruct((1, 1), jnp.float32),
    )(sums_p, cnts_p)
    return out[0, 0]


def kernel(node_pos, edge_index, batch_ids):
    bi = batch_ids.astype(jnp.int32)
    tbl = jnp.concatenate(
        [
            node_pos,
            bi.astype(jnp.float32)[:, None],
            jnp.zeros((N_NODES, W - 4), jnp.float32),
        ],
        axis=1,
    )
    ei = edge_index.astype(jnp.int32)
    sums_p, cnts_p = _edge_var_sc(tbl, ei)
    return _finalize_tc(sums_p, cnts_p)


# speedup vs baseline: 2.6563x; 1.0953x over previous
import functools

import jax
import jax.numpy as jnp
from jax import lax
from jax.experimental import pallas as pl
from jax.experimental.pallas import tpu as pltpu
from jax.experimental.pallas import tpu_sc as plsc

N_NODES = 100000
N_EDGES = 6400000
NUM_GRAPHS = 1024

NC = 2
NS = 16
L = 16
W = 8
NW = NC * NS
EPW = N_EDGES // NW
CHUNK = 2000
NCHUNK = EPW // CHUNK
CVECS = CHUNK // L


def _edge_var_sc(tbl, ei):
    mesh = plsc.VectorSubcoreMesh(
        core_axis_name="c", subcore_axis_name="s", num_cores=NC, num_subcores=NS
    )

    @functools.partial(
        pl.kernel,
        out_type=[
            jax.ShapeDtypeStruct((NW, NUM_GRAPHS), jnp.float32),
            jax.ShapeDtypeStruct((NW, NUM_GRAPHS), jnp.float32),
        ],
        mesh=mesh,
        scratch_types=[
            pltpu.VMEM_SHARED((N_NODES, W), jnp.float32),
            pltpu.VMEM((CHUNK,), jnp.int32),
            pltpu.VMEM((CHUNK,), jnp.int32),
            pltpu.VMEM((CHUNK,), jnp.int32),
            pltpu.VMEM((CHUNK,), jnp.int32),
            pltpu.VMEM((CHUNK, W), jnp.float32),
            pltpu.VMEM((CHUNK, W), jnp.float32),
            pltpu.VMEM((CHUNK, W), jnp.float32),
            pltpu.VMEM((CHUNK, W), jnp.float32),
            pltpu.VMEM((NUM_GRAPHS,), jnp.float32),
            pltpu.VMEM((NUM_GRAPHS,), jnp.float32),
            pltpu.SemaphoreType.DMA,
            pltpu.SemaphoreType.DMA,
            pltpu.SemaphoreType.DMA,
            pltpu.SemaphoreType.DMA,
        ],
        compiler_params=pltpu.CompilerParams(
            needs_layout_passes=False, use_tc_tiling_on_sc=False
        ),
    )
    def body(tbl_hbm, ei_hbm, sums_out, cnts_out,
             tbl_sh, sidx0, didx0, sidx1, didx1, srows0, drows0,
             srows1, drows1, lsum, lcnt, semi0, semi1, semg0, semg1):
        cid = lax.axis_index("c")
        sid = lax.axis_index("s")
        wid = cid * NS + sid

        @pl.when(sid == 0)
        def _():
            pltpu.sync_copy(tbl_hbm, tbl_sh)

        def zbody(i, _):
            off = pl.multiple_of(i * L, L)
            lsum[pl.ds(off, L)] = jnp.zeros((L,), jnp.float32)
            lcnt[pl.ds(off, L)] = jnp.zeros((L,), jnp.float32)
            return 0

        lax.fori_loop(0, NUM_GRAPHS // L, zbody, 0)
        plsc.subcore_barrier()

        ones = jnp.ones((L,), jnp.float32)
        c0 = jnp.zeros((L,), jnp.int32)
        c1 = jnp.full((L,), 1, jnp.int32)
        c2 = jnp.full((L,), 2, jnp.int32)
        c3 = jnp.full((L,), 3, jnp.int32)

        def compute(srows, drows):
            def edge_block(rows):
                sx = plsc.load_gather(srows, [rows, c0])
                sy = plsc.load_gather(srows, [rows, c1])
                sz = plsc.load_gather(srows, [rows, c2])
                bf = plsc.load_gather(srows, [rows, c3])
                dx = plsc.load_gather(drows, [rows, c0])
                dy = plsc.load_gather(drows, [rows, c1])
                dz = plsc.load_gather(drows, [rows, c2])
                ex = dx - sx
                ey = dy - sy
                ez = dz - sz
                s = ex * ex + ey * ey + ez * ez + jnp.float32(1e-12)
                bits = plsc.bitcast(s, jnp.int32)
                bits = jnp.int32(0x5F3759DF) - lax.shift_right_logical(bits, 1)
                y = plsc.bitcast(bits, jnp.float32)
                half = s * jnp.float32(0.5)
                for _ in range(2):
                    y = y * (jnp.float32(1.5) - half * y * y)
                eu = s * y
                d = eu - jnp.float32(1.0)
                var = d * d
                bidx = bf.astype(jnp.int32)
                plsc.addupdate_scatter(lsum, [bidx], var)
                plsc.addupdate_scatter(lcnt, [bidx], ones)

            def vec_body(vi, _):
                base = vi * (2 * L) + lax.iota(jnp.int32, L)
                edge_block(base)
                edge_block(base + L)
                return 0

            lax.fori_loop(0, CVECS // 2, vec_body, 0)

        def launch_idx(ci, sidx, didx, semi):
            base = pl.multiple_of(wid * EPW + ci * CHUNK, 8)
            pltpu.async_copy(ei_hbm.at[0, pl.ds(base, CHUNK)], sidx, semi)
            pltpu.async_copy(ei_hbm.at[1, pl.ds(base, CHUNK)], didx, semi)

        def wait_idx(sidx, didx, semi):
            pltpu.make_async_copy(ei_hbm.at[0, pl.ds(0, CHUNK)], sidx, semi).wait()
            pltpu.make_async_copy(ei_hbm.at[1, pl.ds(0, CHUNK)], didx, semi).wait()

        def launch_gather(sidx, didx, srows, drows, semg):
            pltpu.async_copy(tbl_sh.at[sidx], srows, semg)
            pltpu.async_copy(tbl_sh.at[didx], drows, semg)

        def wait_gather(sidx, didx, srows, drows, semg):
            pltpu.make_async_copy(tbl_hbm.at[sidx], srows, semg).wait()
            pltpu.make_async_copy(tbl_hbm.at[didx], drows, semg).wait()

        launch_idx(0, sidx0, didx0, semi0)
        wait_idx(sidx0, didx0, semi0)
        launch_gather(sidx0, didx0, srows0, drows0, semg0)
        launch_idx(1, sidx1, didx1, semi1)

        def pair_body(g, _):
            ca = 2 * g
            wait_gather(sidx0, didx0, srows0, drows0, semg0)
            wait_idx(sidx1, didx1, semi1)
            launch_gather(sidx1, didx1, srows1, drows1, semg1)

            @pl.when(ca + 2 < NCHUNK)
            def _():
                launch_idx(ca + 2, sidx0, didx0, semi0)

            compute(srows0, drows0)

            cb = ca + 1
            wait_gather(sidx1, didx1, srows1, drows1, semg1)

            @pl.when(cb + 1 < NCHUNK)
            def _():
                wait_idx(sidx0, didx0, semi0)
                launch_gather(sidx0, didx0, srows0, drows0, semg0)

            @pl.when(cb + 2 < NCHUNK)
            def _():
                launch_idx(cb + 2, sidx1, didx1, semi1)

            compute(srows1, drows1)
            return 0

        lax.fori_loop(0, NCHUNK // 2, pair_body, 0)

        pltpu.sync_copy(lsum, sums_out.at[wid])
        pltpu.sync_copy(lcnt, cnts_out.at[wid])

    return body(tbl, ei)


def _finalize_tc(sums_p, cnts_p):
    def tc_body(s_ref, c_ref, o_ref):
        s = jnp.sum(s_ref[...], axis=0)
        c = jnp.sum(c_ref[...], axis=0)
        gv = s / jnp.maximum(c, 1.0)
        o_ref[...] = (jnp.sum(gv) / jnp.float32(NUM_GRAPHS)).reshape(1, 1)

    out = pl.pallas_call(
        tc_body,
        out_shape=jax.ShapeDtypeStruct((1, 1), jnp.float32),
    )(sums_p, cnts_p)
    return out[0, 0]


def kernel(node_pos, edge_index, batch_ids):
    bi = batch_ids.astype(jnp.int32)
    tbl = jnp.concatenate(
        [
            node_pos,
            bi.astype(jnp.float32)[:, None],
            jnp.zeros((N_NODES, W - 4), jnp.float32),
        ],
        axis=1,
    )
    ei = edge_index.astype(jnp.int32)
    sums_p, cnts_p = _edge_var_sc(tbl, ei)
    return _finalize_tc(sums_p, cnts_p)
